# baseline (speedup 1.0000x reference)
import functools

import jax
import jax.numpy as jnp
from jax import lax
from jax.experimental import pallas as pl
from jax.experimental.pallas import tpu as pltpu
from jax.experimental.pallas import tpu_sc as plsc

N = 10000
E = 320000
IN = 128
H = 8
D = 16
FD = H * D
HID = 128
OUT = 8

NC = 2
NS = 16
NW = NC * NS
ROW = FD + 16
N_PAD = 10112
RPT = N_PAD // NS
CH = 104
E_T = 10192
E_PAD = NW * E_T
CHUNKS = E_T // CH
HPAIR = CHUNKS // 2


def _pre_body(h_ref, w0_ref, al0_ref, ar0_ref, w1_ref, al1_ref, ar1_ref,
              f0_ref, e0_ref, f1_ref, e1_ref):
    gi = lax.broadcasted_iota(jnp.int32, (FD, 16), 0) // D
    ji = lax.broadcasted_iota(jnp.int32, (FD, 16), 1)
    mask16 = (gi == ji).astype(jnp.float32)
    hh = h_ref[...]
    for w_ref, al_ref, ar_ref, f_ref, e_ref in (
        (w0_ref, al0_ref, ar0_ref, f0_ref, e0_ref),
        (w1_ref, al1_ref, ar1_ref, f1_ref, e1_ref),
    ):
        feat = jnp.dot(hh, w_ref[...], preferred_element_type=jnp.float32)
        el16 = jnp.dot(feat * al_ref[...], mask16,
                       preferred_element_type=jnp.float32)
        er16 = jnp.dot(feat * ar_ref[...], mask16,
                       preferred_element_type=jnp.float32)
        f_ref[...] = jnp.zeros((N_PAD, ROW), jnp.float32)
        f_ref[0:N, 0:FD] = feat
        f_ref[0:N, FD:ROW] = el16
        e_ref[...] = jnp.zeros((N_PAD, 16), jnp.float32)
        e_ref[0:N, :] = er16


def _sc_body(f0_ref, e0_ref, sd0_ref, f1_ref, e1_ref, sd1_ref, zc_ref,
             o0_ref, o1_ref,
             acc_s, sd_a, sd_b, rows_a, rows_b, er_a, er_b,
             sem_a, sem_b, sem_s):
    cid = lax.axis_index("c")
    sid = lax.axis_index("s")
    wid = cid * NS + sid

    for fl, er, sdr, outr in (
        (f0_ref, e0_ref, sd0_ref, o0_ref),
        (f1_ref, e1_ref, sd1_ref, o1_ref),
    ):
        pltpu.sync_copy(zc_ref, acc_s.at[pl.ds(sid * RPT, RPT)])
        plsc.subcore_barrier()

        def idxload(c, sd):
            pltpu.sync_copy(sdr.at[wid * CHUNKS + c], sd)

        def gather(sd, rows, erv, sem):
            pltpu.async_copy(fl.at[sd.at[0]], rows, sem)
            pltpu.async_copy(er.at[sd.at[1]], erv, sem)

        def waitg(sd, rows, erv, sem):
            pltpu.make_async_copy(fl.at[sd.at[0]], rows, sem).wait()
            pltpu.make_async_copy(er.at[sd.at[1]], erv, sem).wait()

        lane_j = [jnp.full((16,), j, jnp.int32) for j in range(H)]

        def compute(rows, erv):
            @plsc.parallel_loop(0, CH, 1, unroll=4)
            def _edge(e):
                el16 = rows[e, pl.ds(FD, 16)]
                er16 = erv[e, :]
                fs = [rows[e, pl.ds(j * D, 16)] for j in range(H)]
                x = el16 + er16
                x = jnp.maximum(x, 0.2 * x)
                w = jnp.exp(x)
                rows[e, pl.ds(FD, 16)] = w
                for j in range(H):
                    wj = jnp.take(w, lane_j[j])
                    rows[e, pl.ds(j * D, 16)] = fs[j] * wj

        idxload(0, sd_a)
        gather(sd_a, rows_a, er_a, sem_a)

        def pair_body(i, carry):
            c_a = 2 * i
            c_b = c_a + 1
            idxload(c_b, sd_b)
            gather(sd_b, rows_b, er_b, sem_b)
            waitg(sd_a, rows_a, er_a, sem_a)
            compute(rows_a, er_a)
            pltpu.async_copy(rows_a, acc_s.at[sd_a.at[1]], sem_s, add=True)

            waitg(sd_b, rows_b, er_b, sem_b)
            compute(rows_b, er_b)
            pltpu.make_async_copy(rows_a, acc_s.at[sd_a.at[1]], sem_s).wait()

            @pl.when(i < HPAIR - 1)
            def _():
                idxload(c_a + 2, sd_a)
                gather(sd_a, rows_a, er_a, sem_a)

            pltpu.sync_copy(rows_b, acc_s.at[sd_b.at[1]], add=True)
            return carry

        lax.fori_loop(0, HPAIR, pair_body, 0)
        plsc.subcore_barrier()
        pltpu.sync_copy(acc_s.at[pl.ds(sid * RPT, RPT)],
                        outr.at[cid, pl.ds(sid * RPT, RPT)])
        plsc.subcore_barrier()


def _post_body(a0_ref, a1_ref, b0_ref, b1_ref, sw1_ref, sb1_ref, sw2_ref,
               pw_ref, pb_ref, o_ref):
    hi = lax.broadcasted_iota(jnp.int32, (H, FD), 0)
    gi = lax.broadcasted_iota(jnp.int32, (H, FD), 1) // D
    expand = (hi == gi).astype(jnp.float32)
    outs = []
    for a_ref, b_ref in ((a0_ref, b0_ref), (a1_ref, b1_ref)):
        a = a_ref[0] + a_ref[1]
        f = a[0:N, 0:FD]
        s8 = a[0:N, FD:FD + H]
        s16 = jnp.dot(s8, expand, preferred_element_type=jnp.float32)
        z = f / (s16 + 1e-9) + b_ref[...]
        z = jnp.where(z > 0, z, jnp.exp(jnp.minimum(z, 0.0)) - 1.0)
        t = jnp.tanh(jnp.dot(z, sw1_ref[...],
                             preferred_element_type=jnp.float32)
                     + sb1_ref[...])
        wv = jnp.sum(t * sw2_ref[...], axis=1, keepdims=True)
        ew = jnp.exp(wv - jnp.max(wv))
        agg = jnp.sum(z * ew, axis=0, keepdims=True) / jnp.sum(ew)
        outs.append(jnp.dot(agg, pw_ref[...],
                            preferred_element_type=jnp.float32)
                    + pb_ref[...])
    o_ref[...] = jnp.concatenate(outs, axis=0)


@jax.jit
def kernel(h, edge_index_0, edge_index_1, W0, al0, ar0, b0,
           W1, al1, ar1, b1, sW1, sb1, sW2, pW, pb):
    f32 = jnp.float32

    featl0, er0, featl1, er1 = pl.pallas_call(
        _pre_body,
        out_shape=[
            jax.ShapeDtypeStruct((N_PAD, ROW), f32),
            jax.ShapeDtypeStruct((N_PAD, 16), f32),
            jax.ShapeDtypeStruct((N_PAD, ROW), f32),
            jax.ShapeDtypeStruct((N_PAD, 16), f32),
        ],
    )(h, W0, al0.reshape(1, FD), ar0.reshape(1, FD),
      W1, al1.reshape(1, FD), ar1.reshape(1, FD))

    pad = jnp.full((2, E_PAD - E), N, jnp.int32)
    sd0 = jnp.concatenate([edge_index_0, pad], axis=1) \
        .reshape(2, E_PAD // CH, CH).transpose(1, 0, 2)
    sd1 = jnp.concatenate([edge_index_1, pad], axis=1) \
        .reshape(2, E_PAD // CH, CH).transpose(1, 0, 2)
    zc = jnp.zeros((RPT, ROW), f32)

    mesh = plsc.VectorSubcoreMesh(core_axis_name="c", subcore_axis_name="s",
                                  num_cores=NC, num_subcores=NS)
    acc0, acc1 = pl.kernel(
        _sc_body,
        out_type=[
            jax.ShapeDtypeStruct((NC, N_PAD, ROW), f32),
            jax.ShapeDtypeStruct((NC, N_PAD, ROW), f32),
        ],
        mesh=mesh,
        compiler_params=pltpu.CompilerParams(use_tc_tiling_on_sc=False),
        scratch_types=[
            pltpu.VMEM_SHARED((N_PAD, ROW), f32),
            pltpu.VMEM((2, CH), jnp.int32),
            pltpu.VMEM((2, CH), jnp.int32),
            pltpu.VMEM((CH, ROW), f32),
            pltpu.VMEM((CH, ROW), f32),
            pltpu.VMEM((CH, 16), f32),
            pltpu.VMEM((CH, 16), f32),
            pltpu.SemaphoreType.DMA,
            pltpu.SemaphoreType.DMA,
            pltpu.SemaphoreType.DMA,
        ],
    )(featl0, er0, sd0, featl1, er1, sd1, zc)

    out = pl.pallas_call(
        _post_body,
        out_shape=jax.ShapeDtypeStruct((2, OUT), f32),
    )(acc0, acc1, b0.reshape(1, FD), b1.reshape(1, FD), sW1,
      sb1.reshape(1, HID), sW2.reshape(1, HID), pW, pb.reshape(1, OUT))
    return out


# speedup vs baseline: 1.0877x; 1.0877x over previous
import functools

import jax
import jax.numpy as jnp
from jax import lax
from jax.experimental import pallas as pl
from jax.experimental.pallas import tpu as pltpu
from jax.experimental.pallas import tpu_sc as plsc

N = 10000
E = 320000
IN = 128
H = 8
D = 16
FD = H * D
HID = 128
OUT = 8

NC = 2
NS = 16
NW = NC * NS
ROW = FD + 16
N_PAD = 10112
RPT = N_PAD // NS
CH = 104
E_T = 10192
E_PAD = NW * E_T
CHUNKS = E_T // CH
HPAIR = CHUNKS // 2


def _pre_body(h_ref, w0_ref, al0_ref, ar0_ref, w1_ref, al1_ref, ar1_ref,
              f0_ref, e0_ref, f1_ref, e1_ref):
    gi = lax.broadcasted_iota(jnp.int32, (FD, 16), 0) // D
    ji = lax.broadcasted_iota(jnp.int32, (FD, 16), 1)
    mask16 = (gi == ji).astype(jnp.float32)
    hh = h_ref[...]
    for w_ref, al_ref, ar_ref, f_ref, e_ref in (
        (w0_ref, al0_ref, ar0_ref, f0_ref, e0_ref),
        (w1_ref, al1_ref, ar1_ref, f1_ref, e1_ref),
    ):
        feat = jnp.dot(hh, w_ref[...], preferred_element_type=jnp.float32)
        el16 = jnp.dot(feat * al_ref[...], mask16,
                       preferred_element_type=jnp.float32)
        er16 = jnp.dot(feat * ar_ref[...], mask16,
                       preferred_element_type=jnp.float32)
        f_ref[...] = jnp.zeros((N_PAD, ROW), jnp.float32)
        f_ref[0:N, 0:FD] = feat
        f_ref[0:N, FD:ROW] = el16
        e_ref[...] = jnp.zeros((N_PAD, 16), jnp.float32)
        e_ref[0:N, :] = er16


def _sc_body(f0_ref, e0_ref, sd0_ref, f1_ref, e1_ref, sd1_ref, zc_ref,
             o0_ref, o1_ref,
             acc_s, sd_a, sd_b, rows_a, rows_b, er_a, er_b, sem_a, sem_b):
    cid = lax.axis_index("c")
    sid = lax.axis_index("s")
    wid = cid * NS + sid

    for fl, er, sdr, outr in (
        (f0_ref, e0_ref, sd0_ref, o0_ref),
        (f1_ref, e1_ref, sd1_ref, o1_ref),
    ):
        pltpu.sync_copy(zc_ref, acc_s.at[pl.ds(sid * RPT, RPT)])
        plsc.subcore_barrier()

        def idxload(c, sd):
            pltpu.sync_copy(sdr.at[wid * CHUNKS + c], sd)

        def gather(sd, rows, erv, sem):
            pltpu.async_copy(fl.at[sd.at[0]], rows, sem)
            pltpu.async_copy(er.at[sd.at[1]], erv, sem)

        def waitg(sd, rows, erv, sem):
            pltpu.make_async_copy(fl.at[sd.at[0]], rows, sem).wait()
            pltpu.make_async_copy(er.at[sd.at[1]], erv, sem).wait()

        lane_j = [jnp.full((16,), j, jnp.int32) for j in range(H)]

        def compute(rows, erv):
            @plsc.parallel_loop(0, CH, 1, unroll=4)
            def _edge(e):
                el16 = rows[e, pl.ds(FD, 16)]
                er16 = erv[e, :]
                fs = [rows[e, pl.ds(j * D, 16)] for j in range(H)]
                x = el16 + er16
                x = jnp.maximum(x, 0.2 * x)
                w = jnp.exp(x)
                rows[e, pl.ds(FD, 16)] = w
                for j in range(H):
                    wj = jnp.take(w, lane_j[j])
                    rows[e, pl.ds(j * D, 16)] = fs[j] * wj

        idxload(0, sd_a)
        gather(sd_a, rows_a, er_a, sem_a)

        def pair_body(i, carry):
            c_a = 2 * i
            c_b = c_a + 1
            idxload(c_b, sd_b)
            gather(sd_b, rows_b, er_b, sem_b)
            waitg(sd_a, rows_a, er_a, sem_a)
            compute(rows_a, er_a)
            pltpu.sync_copy(rows_a, acc_s.at[sd_a.at[1]], add=True)

            @pl.when(i < HPAIR - 1)
            def _():
                idxload(c_a + 2, sd_a)
                gather(sd_a, rows_a, er_a, sem_a)

            waitg(sd_b, rows_b, er_b, sem_b)
            compute(rows_b, er_b)
            pltpu.sync_copy(rows_b, acc_s.at[sd_b.at[1]], add=True)
            return carry

        lax.fori_loop(0, HPAIR, pair_body, 0)
        plsc.subcore_barrier()
        pltpu.sync_copy(acc_s.at[pl.ds(sid * RPT, RPT)],
                        outr.at[cid, pl.ds(sid * RPT, RPT)])
        plsc.subcore_barrier()


def _post_body(a0_ref, a1_ref, b0_ref, b1_ref, sw1_ref, sb1_ref, sw2_ref,
               pw_ref, pb_ref, o_ref):
    hi = lax.broadcasted_iota(jnp.int32, (H, FD), 0)
    gi = lax.broadcasted_iota(jnp.int32, (H, FD), 1) // D
    expand = (hi == gi).astype(jnp.float32)
    outs = []
    for a_ref, b_ref in ((a0_ref, b0_ref), (a1_ref, b1_ref)):
        a = a_ref[0] + a_ref[1]
        f = a[0:N, 0:FD]
        s8 = a[0:N, FD:FD + H]
        s16 = jnp.dot(s8, expand, preferred_element_type=jnp.float32)
        z = f / (s16 + 1e-9) + b_ref[...]
        z = jnp.where(z > 0, z, jnp.exp(jnp.minimum(z, 0.0)) - 1.0)
        t = jnp.tanh(jnp.dot(z, sw1_ref[...],
                             preferred_element_type=jnp.float32)
                     + sb1_ref[...])
        wv = jnp.sum(t * sw2_ref[...], axis=1, keepdims=True)
        ew = jnp.exp(wv - jnp.max(wv))
        agg = jnp.sum(z * ew, axis=0, keepdims=True) / jnp.sum(ew)
        outs.append(jnp.dot(agg, pw_ref[...],
                            preferred_element_type=jnp.float32)
                    + pb_ref[...])
    o_ref[...] = jnp.concatenate(outs, axis=0)


@jax.jit
def kernel(h, edge_index_0, edge_index_1, W0, al0, ar0, b0,
           W1, al1, ar1, b1, sW1, sb1, sW2, pW, pb):
    f32 = jnp.float32

    featl0, er0, featl1, er1 = pl.pallas_call(
        _pre_body,
        out_shape=[
            jax.ShapeDtypeStruct((N_PAD, ROW), f32),
            jax.ShapeDtypeStruct((N_PAD, 16), f32),
            jax.ShapeDtypeStruct((N_PAD, ROW), f32),
            jax.ShapeDtypeStruct((N_PAD, 16), f32),
        ],
    )(h, W0, al0.reshape(1, FD), ar0.reshape(1, FD),
      W1, al1.reshape(1, FD), ar1.reshape(1, FD))

    pad = jnp.full((2, E_PAD - E), N, jnp.int32)
    sd0 = jnp.concatenate([edge_index_0, pad], axis=1) \
        .reshape(2, E_PAD // CH, CH).transpose(1, 0, 2)
    sd1 = jnp.concatenate([edge_index_1, pad], axis=1) \
        .reshape(2, E_PAD // CH, CH).transpose(1, 0, 2)
    zc = jnp.zeros((RPT, ROW), f32)

    mesh = plsc.VectorSubcoreMesh(core_axis_name="c", subcore_axis_name="s",
                                  num_cores=NC, num_subcores=NS)
    acc0, acc1 = pl.kernel(
        _sc_body,
        out_type=[
            jax.ShapeDtypeStruct((NC, N_PAD, ROW), f32),
            jax.ShapeDtypeStruct((NC, N_PAD, ROW), f32),
        ],
        mesh=mesh,
        compiler_params=pltpu.CompilerParams(use_tc_tiling_on_sc=False),
        scratch_types=[
            pltpu.VMEM_SHARED((N_PAD, ROW), f32),
            pltpu.VMEM((2, CH), jnp.int32),
            pltpu.VMEM((2, CH), jnp.int32),
            pltpu.VMEM((CH, ROW), f32),
            pltpu.VMEM((CH, ROW), f32),
            pltpu.VMEM((CH, 16), f32),
            pltpu.VMEM((CH, 16), f32),
            pltpu.SemaphoreType.DMA,
            pltpu.SemaphoreType.DMA,
        ],
    )(featl0, er0, sd0, featl1, er1, sd1, zc)

    out = pl.pallas_call(
        _post_body,
        out_shape=jax.ShapeDtypeStruct((2, OUT), f32),
    )(acc0, acc1, b0.reshape(1, FD), b1.reshape(1, FD), sW1,
      sb1.reshape(1, HID), sW2.reshape(1, HID), pW, pb.reshape(1, OUT))
    return out


# speedup vs baseline: 1.1406x; 1.0486x over previous
import functools

import jax
import jax.numpy as jnp
from jax import lax
from jax.experimental import pallas as pl
from jax.experimental.pallas import tpu as pltpu
from jax.experimental.pallas import tpu_sc as plsc

N = 10000
E = 320000
IN = 128
H = 8
D = 16
FD = H * D
HID = 128
OUT = 8

NC = 2
NS = 16
NW = NC * NS
ROW = FD + 16
N_PAD = 10112
RPT = N_PAD // NS
CH = 96
E_T = 10176
E_PAD = NW * E_T
CHUNKS = E_T // CH
HPAIR = CHUNKS // 2


def _pre_body(h_ref, w0_ref, al0_ref, ar0_ref, w1_ref, al1_ref, ar1_ref,
              f0_ref, e0_ref, f1_ref, e1_ref):
    gi = lax.broadcasted_iota(jnp.int32, (FD, 16), 0) // D
    ji = lax.broadcasted_iota(jnp.int32, (FD, 16), 1)
    mask16 = (gi == ji).astype(jnp.float32)
    hh = h_ref[...]
    for w_ref, al_ref, ar_ref, f_ref, e_ref in (
        (w0_ref, al0_ref, ar0_ref, f0_ref, e0_ref),
        (w1_ref, al1_ref, ar1_ref, f1_ref, e1_ref),
    ):
        feat = jnp.dot(hh, w_ref[...], preferred_element_type=jnp.float32)
        el16 = jnp.dot(feat * al_ref[...], mask16,
                       preferred_element_type=jnp.float32)
        er16 = jnp.dot(feat * ar_ref[...], mask16,
                       preferred_element_type=jnp.float32)
        f_ref[...] = jnp.zeros((N_PAD, ROW), jnp.float32)
        f_ref[0:N, 0:FD] = feat
        f_ref[0:N, FD:ROW] = el16
        e_ref[...] = jnp.zeros((N_PAD, 16), jnp.float32)
        e_ref[0:N, :] = er16


def _sc_body(f0_ref, e0_ref, sd0_ref, f1_ref, e1_ref, sd1_ref, zc_ref,
             o0_ref, o1_ref,
             acc_s, sd_a, sd_b, rows_a, rows_b, er_a, er_b, sem_a, sem_b):
    cid = lax.axis_index("c")
    sid = lax.axis_index("s")
    wid = cid * NS + sid

    for fl, er, sdr, outr in (
        (f0_ref, e0_ref, sd0_ref, o0_ref),
        (f1_ref, e1_ref, sd1_ref, o1_ref),
    ):
        pltpu.sync_copy(zc_ref, acc_s.at[pl.ds(sid * RPT, RPT)])
        plsc.subcore_barrier()

        def idxload(c, sd):
            pltpu.sync_copy(sdr.at[wid * CHUNKS + c], sd)

        def gather(sd, rows, erv, sem):
            pltpu.async_copy(fl.at[sd.at[0]], rows, sem)
            pltpu.async_copy(er.at[sd.at[1]], erv, sem)

        def waitg(sd, rows, erv, sem):
            pltpu.make_async_copy(fl.at[sd.at[0]], rows, sem).wait()
            pltpu.make_async_copy(er.at[sd.at[1]], erv, sem).wait()

        lane_j = [jnp.full((16,), j, jnp.int32) for j in range(H)]

        def compute(rows, erv):
            @plsc.parallel_loop(0, CH, 1, unroll=4)
            def _edge(e):
                el16 = rows[e, pl.ds(FD, 16)]
                er16 = erv[e, :]
                fs = [rows[e, pl.ds(j * D, 16)] for j in range(H)]
                x = el16 + er16
                x = jnp.maximum(x, 0.2 * x)
                w = jnp.exp(x)
                rows[e, pl.ds(FD, 16)] = w
                for j in range(H):
                    wj = jnp.take(w, lane_j[j])
                    rows[e, pl.ds(j * D, 16)] = fs[j] * wj

        idxload(0, sd_a)
        gather(sd_a, rows_a, er_a, sem_a)

        def pair_body(i, carry):
            c_a = 2 * i
            c_b = c_a + 1
            idxload(c_b, sd_b)
            gather(sd_b, rows_b, er_b, sem_b)
            waitg(sd_a, rows_a, er_a, sem_a)
            compute(rows_a, er_a)
            pltpu.sync_copy(rows_a, acc_s.at[sd_a.at[1]], add=True)

            @pl.when(i < HPAIR - 1)
            def _():
                idxload(c_a + 2, sd_a)
                gather(sd_a, rows_a, er_a, sem_a)

            waitg(sd_b, rows_b, er_b, sem_b)
            compute(rows_b, er_b)
            pltpu.sync_copy(rows_b, acc_s.at[sd_b.at[1]], add=True)
            return carry

        lax.fori_loop(0, HPAIR, pair_body, 0)
        plsc.subcore_barrier()
        pltpu.sync_copy(acc_s.at[pl.ds(sid * RPT, RPT)],
                        outr.at[cid, pl.ds(sid * RPT, RPT)])
        plsc.subcore_barrier()


def _post_body(a0_ref, a1_ref, b0_ref, b1_ref, sw1_ref, sb1_ref, sw2_ref,
               pw_ref, pb_ref, o_ref):
    hi = lax.broadcasted_iota(jnp.int32, (H, FD), 0)
    gi = lax.broadcasted_iota(jnp.int32, (H, FD), 1) // D
    expand = (hi == gi).astype(jnp.float32)
    outs = []
    for a_ref, b_ref in ((a0_ref, b0_ref), (a1_ref, b1_ref)):
        a = a_ref[0] + a_ref[1]
        f = a[0:N, 0:FD]
        s8 = a[0:N, FD:FD + H]
        s16 = jnp.dot(s8, expand, preferred_element_type=jnp.float32)
        z = f / (s16 + 1e-9) + b_ref[...]
        z = jnp.where(z > 0, z, jnp.exp(jnp.minimum(z, 0.0)) - 1.0)
        t = jnp.tanh(jnp.dot(z, sw1_ref[...],
                             preferred_element_type=jnp.float32)
                     + sb1_ref[...])
        wv = jnp.sum(t * sw2_ref[...], axis=1, keepdims=True)
        ew = jnp.exp(wv - jnp.max(wv))
        agg = jnp.sum(z * ew, axis=0, keepdims=True) / jnp.sum(ew)
        outs.append(jnp.dot(agg, pw_ref[...],
                            preferred_element_type=jnp.float32)
                    + pb_ref[...])
    o_ref[...] = jnp.concatenate(outs, axis=0)


@jax.jit
def kernel(h, edge_index_0, edge_index_1, W0, al0, ar0, b0,
           W1, al1, ar1, b1, sW1, sb1, sW2, pW, pb):
    f32 = jnp.float32

    featl0, er0, featl1, er1 = pl.pallas_call(
        _pre_body,
        out_shape=[
            jax.ShapeDtypeStruct((N_PAD, ROW), f32),
            jax.ShapeDtypeStruct((N_PAD, 16), f32),
            jax.ShapeDtypeStruct((N_PAD, ROW), f32),
            jax.ShapeDtypeStruct((N_PAD, 16), f32),
        ],
    )(h, W0, al0.reshape(1, FD), ar0.reshape(1, FD),
      W1, al1.reshape(1, FD), ar1.reshape(1, FD))

    pad = jnp.full((2, E_PAD - E), N, jnp.int32)
    sd0 = jnp.concatenate([edge_index_0, pad], axis=1) \
        .reshape(2, E_PAD // CH, CH).transpose(1, 0, 2)
    sd1 = jnp.concatenate([edge_index_1, pad], axis=1) \
        .reshape(2, E_PAD // CH, CH).transpose(1, 0, 2)
    zc = jnp.zeros((RPT, ROW), f32)

    mesh = plsc.VectorSubcoreMesh(core_axis_name="c", subcore_axis_name="s",
                                  num_cores=NC, num_subcores=NS)
    acc0, acc1 = pl.kernel(
        _sc_body,
        out_type=[
            jax.ShapeDtypeStruct((NC, N_PAD, ROW), f32),
            jax.ShapeDtypeStruct((NC, N_PAD, ROW), f32),
        ],
        mesh=mesh,
        compiler_params=pltpu.CompilerParams(use_tc_tiling_on_sc=False),
        scratch_types=[
            pltpu.VMEM_SHARED((N_PAD, ROW), f32),
            pltpu.VMEM((2, CH), jnp.int32),
            pltpu.VMEM((2, CH), jnp.int32),
            pltpu.VMEM((CH, ROW), f32),
            pltpu.VMEM((CH, ROW), f32),
            pltpu.VMEM((CH, 16), f32),
            pltpu.VMEM((CH, 16), f32),
            pltpu.SemaphoreType.DMA,
            pltpu.SemaphoreType.DMA,
        ],
    )(featl0, er0, sd0, featl1, er1, sd1, zc)

    out = pl.pallas_call(
        _post_body,
        out_shape=jax.ShapeDtypeStruct((2, OUT), f32),
    )(acc0, acc1, b0.reshape(1, FD), b1.reshape(1, FD), sW1,
      sb1.reshape(1, HID), sW2.reshape(1, HID), pW, pb.reshape(1, OUT))
    return out
